# baseline (speedup 1.0000x reference)
import functools

import jax
import jax.numpy as jnp
from jax import lax
from jax.experimental import pallas as pl
from jax.experimental.pallas import tpu as pltpu
from jax.experimental.pallas import tpu_sc as plsc

N = 10000
E = 320000
D = 128

NC = 2
NS = 16
NW = NC * NS
K = 120
C = 84
CP = C + 3
KC = 128
CC = 80
CPC = CC + 4

NPAD = 10240
ROWS_PER_TILE = NPAD // NS
OB = 80
OUT_CHUNKS = ROWS_PER_TILE // OB

_MESH = plsc.VectorSubcoreMesh(core_axis_name="c", subcore_axis_name="s")


def _fill_rows(buf, val, nrows=K):
    v16 = jnp.full((16,), val, jnp.float32)

    @pl.loop(0, nrows)
    def _f(i):
        for j in range(D // 16):
            buf[i, pl.ds(j * 16, 16)] = v16


def _zero_acc_slice(zbuf, acc, sid):
    @pl.loop(0, OUT_CHUNKS)
    def _zacc(k):
        pltpu.sync_copy(zbuf.at[pl.ds(0, OB)],
                        acc.at[pl.ds(sid * ROWS_PER_TILE + k * OB, OB)])


def _copy_out_slice(buf, acc, out_hbm, cid, sid):
    @pl.loop(0, OUT_CHUNKS)
    def _out(k):
        r0 = sid * ROWS_PER_TILE + k * OB
        pltpu.sync_copy(acc.at[pl.ds(r0, OB)], buf.at[pl.ds(0, OB)])
        pltpu.sync_copy(buf.at[pl.ds(0, OB)],
                        out_hbm.at[pl.ds(cid * NPAD + r0, OB)])


def _sc_agg_body(x_hbm, src_hbm, dst_hbm, agg_out, *, srcb, idab, rows0,
                 rows1, rows2, acc, g0, g1, g2, s0, s1, s2,
                 i0, i1, i2, i3, j0, j1, j2, j3):
    cid = lax.axis_index("c")
    sid = lax.axis_index("s")
    wid = sid * NC + cid
    bufs = (rows0, rows1, rows2)
    gsems = (g0, g1, g2)
    ssems = (s0, s1, s2)
    isems = (i0, i1, i2, i3)
    jsems = (j0, j1, j2, j3)
    ebase = wid * (CP * K)

    def src_cp(c, slot):
        return pltpu.make_async_copy(
            src_hbm.at[pl.ds(ebase + c * K, K)], srcb.at[slot], isems[slot])

    def dst_cp(c, slot):
        return pltpu.make_async_copy(
            dst_hbm.at[pl.ds(ebase + c * K, K)], idab.at[slot], jsems[slot])

    _fill_rows(rows0, 0.0)
    _fill_rows(rows1, 0.0)
    _fill_rows(rows2, 0.0)

    _zero_acc_slice(rows0, acc, sid)

    plsc.subcore_barrier()

    for t in range(3):
        src_cp(t, t).start()
        dst_cp(t, t).start()
    src_cp(0, 0).wait()
    pltpu.async_copy(x_hbm.at[srcb.at[0]], rows0, g0)
    src_cp(1, 1).wait()
    pltpu.async_copy(x_hbm.at[srcb.at[1]], rows1, g1)
    pltpu.async_copy(rows2,
                     agg_out.at[pl.ds(cid * NPAD + sid * ROWS_PER_TILE, K)],
                     s2)

    @pl.loop(0, C, step=12)
    def _chunk(cc):
        for u in range(12):
            c = cc + u
            rb = u % 3
            pltpu.make_async_copy(
                x_hbm.at[srcb.at[u % 4]], bufs[rb], gsems[rb]).wait()
            dst_cp(c, u % 4).wait()
            pltpu.async_copy(bufs[rb], acc.at[idab.at[u % 4]], ssems[rb],
                             add=True)
            pltpu.make_async_copy(
                bufs[(u + 2) % 3], acc.at[idab.at[(u + 3) % 4]],
                ssems[(u + 2) % 3]).wait()
            src_cp(c + 3, (u + 3) % 4).start()
            dst_cp(c + 3, (u + 3) % 4).start()
            src_cp(c + 2, (u + 2) % 4).wait()
            pltpu.async_copy(x_hbm.at[srcb.at[(u + 2) % 4]],
                             bufs[(u + 2) % 3], gsems[(u + 2) % 3])

    pltpu.make_async_copy(rows2, acc.at[idab.at[3]],
                          ssems[(C - 1) % 3]).wait()
    pltpu.make_async_copy(x_hbm.at[srcb.at[0]], bufs[C % 3],
                          gsems[C % 3]).wait()
    pltpu.make_async_copy(x_hbm.at[srcb.at[0]], bufs[(C + 1) % 3],
                          gsems[(C + 1) % 3]).wait()
    src_cp(C + 2, (C + 2) % 4).wait()
    dst_cp(C, C % 4).wait()
    dst_cp(C + 1, (C + 1) % 4).wait()
    dst_cp(C + 2, (C + 2) % 4).wait()

    plsc.subcore_barrier()

    _copy_out_slice(rows0, acc, agg_out, cid, sid)


_sc_agg = pl.kernel(
    _sc_agg_body,
    out_type=jax.ShapeDtypeStruct((NC * NPAD, D), jnp.float32),
    mesh=_MESH,
    scratch_types=dict(
        srcb=pltpu.VMEM((4, K), jnp.int32),
        idab=pltpu.VMEM((4, K), jnp.int32),
        rows0=pltpu.VMEM((K, D), jnp.float32),
        rows1=pltpu.VMEM((K, D), jnp.float32),
        rows2=pltpu.VMEM((K, D), jnp.float32),
        acc=pltpu.VMEM_SHARED((NPAD, D), jnp.float32),
        g0=pltpu.SemaphoreType.DMA,
        g1=pltpu.SemaphoreType.DMA,
        g2=pltpu.SemaphoreType.DMA,
        s0=pltpu.SemaphoreType.DMA,
        s1=pltpu.SemaphoreType.DMA,
        s2=pltpu.SemaphoreType.DMA,
        i0=pltpu.SemaphoreType.DMA,
        i1=pltpu.SemaphoreType.DMA,
        i2=pltpu.SemaphoreType.DMA,
        i3=pltpu.SemaphoreType.DMA,
        j0=pltpu.SemaphoreType.DMA,
        j1=pltpu.SemaphoreType.DMA,
        j2=pltpu.SemaphoreType.DMA,
        j3=pltpu.SemaphoreType.DMA,
    ))


def _sc_cnt_body(dst_hbm, cnt_out, *, ida, ones_rows, acc, s0, s1, s2, s3):
    cid = lax.axis_index("c")
    sid = lax.axis_index("s")
    wid = sid * NC + cid
    ssems = (s0, s1, s2, s3)

    pltpu.sync_copy(dst_hbm.at[wid], ida)

    _fill_rows(ones_rows, 0.0, KC)

    _zero_acc_slice(ones_rows, acc, sid)

    _fill_rows(ones_rows, 1.0, KC)

    plsc.subcore_barrier()

    for t in range(4):
        pltpu.async_copy(ones_rows, acc.at[ida.at[CC + t]], ssems[t],
                         add=True)

    @pl.loop(0, CC, step=4)
    def _chunk(cc):
        for b in range(4):
            c = cc + b
            pltpu.make_async_copy(
                ones_rows, acc.at[ida.at[c]], ssems[b]).wait()
            pltpu.async_copy(ones_rows, acc.at[ida.at[c]], ssems[b], add=True)

    for t in range(4):
        pltpu.make_async_copy(ones_rows, acc.at[ida.at[0]], ssems[t]).wait()

    plsc.subcore_barrier()

    _copy_out_slice(ones_rows, acc, cnt_out, cid, sid)


_sc_cnt = pl.kernel(
    _sc_cnt_body,
    out_type=jax.ShapeDtypeStruct((NC * NPAD, D), jnp.float32),
    mesh=_MESH,
    scratch_types=dict(
        ida=pltpu.VMEM((CPC, KC), jnp.int32),
        ones_rows=pltpu.VMEM((KC, D), jnp.float32),
        acc=pltpu.VMEM_SHARED((NPAD, D), jnp.float32),
        s0=pltpu.SemaphoreType.DMA,
        s1=pltpu.SemaphoreType.DMA,
        s2=pltpu.SemaphoreType.DMA,
        s3=pltpu.SemaphoreType.DMA,
    ))

BLK = 1024
GRID = NPAD // BLK


def _dense_body(relu, a0_ref, a1_ref, c0_ref, c1_ref, x_ref, wl_ref, wr_ref,
                b_ref, o_ref):
    agg = a0_ref[...] + a1_ref[...]
    cnt = jnp.sum(c0_ref[...] + c1_ref[...], axis=1) * (1.0 / D)
    scale = 1.0 / jnp.maximum(cnt, 1.0)
    mean = agg * scale[:, None]
    dn = (((1,), (1,)), ((), ()))
    h = lax.dot_general(mean, wl_ref[...], dn,
                        preferred_element_type=jnp.float32)
    h = h + lax.dot_general(x_ref[...], wr_ref[...], dn,
                            preferred_element_type=jnp.float32)
    h = h + b_ref[...]
    if relu:
        h = jnp.where(h >= 0, h, 0.01 * h)
    o_ref[...] = h


def _dense(agg_parts, cnt_parts, x, wl, wr, b, relu):
    nb = NPAD // BLK
    return pl.pallas_call(
        functools.partial(_dense_body, relu),
        grid=(GRID,),
        in_specs=[
            pl.BlockSpec((BLK, D), lambda i: (i, 0)),
            pl.BlockSpec((BLK, D), lambda i: (i + nb, 0)),
            pl.BlockSpec((BLK, D), lambda i: (i, 0)),
            pl.BlockSpec((BLK, D), lambda i: (i + nb, 0)),
            pl.BlockSpec((BLK, D), lambda i: (i, 0)),
            pl.BlockSpec((D, D), lambda i: (0, 0)),
            pl.BlockSpec((D, D), lambda i: (0, 0)),
            pl.BlockSpec((1, D), lambda i: (0, 0)),
        ],
        out_specs=pl.BlockSpec((BLK, D), lambda i: (i, 0)),
        out_shape=jax.ShapeDtypeStruct((NPAD, D), jnp.float32),
    )(agg_parts, agg_parts, cnt_parts, cnt_parts, x, wl, wr, b)


def kernel(features, edges, edges2, edge_features, additional_feature,
           W1l, W1r, b1, W2l, W2r, b2):
    src = edges[0].astype(jnp.int32)
    dst = edges[1].astype(jnp.int32)

    def build(ck, cc, cpc, flat):
        pad = NW * cc * ck - E
        pad_src = jnp.arange(pad, dtype=jnp.int32) % N
        pad_dst = N + (jnp.arange(pad, dtype=jnp.int32) % (NPAD - N))
        src_m = jnp.concatenate([src, pad_src]).reshape(NW, cc, ck)
        dst_m = jnp.concatenate([dst, pad_dst]).reshape(NW, cc, ck)
        pf = jnp.arange((cpc - cc) * ck, dtype=jnp.int32)
        src_pf = jnp.broadcast_to((pf % N).reshape(1, cpc - cc, ck),
                                  (NW, cpc - cc, ck))
        dst_pf = jnp.broadcast_to(
            (N + pf % (NPAD - N)).reshape(1, cpc - cc, ck),
            (NW, cpc - cc, ck))
        sp = jnp.concatenate([src_m, src_pf], axis=1)
        dp = jnp.concatenate([dst_m, dst_pf], axis=1)
        if flat:
            sp, dp = sp.reshape(-1), dp.reshape(-1)
        return sp, dp

    src_p, dst_p = build(K, C, CP, True)
    _, dst_c = build(KC, CC, CPC, False)

    x_pad = jnp.pad(features, ((0, NPAD - N), (0, 0)))
    b1r = b1.reshape(1, D)
    b2r = b2.reshape(1, D)

    cnt = _sc_cnt(dst_c)
    agg1 = _sc_agg(features, src_p, dst_p)

    h = _dense(agg1, cnt, x_pad, W1l, W1r, b1r, relu=True)

    agg2 = _sc_agg(h, src_p, dst_p)

    out = _dense(agg2, cnt, h, W2l, W2r, b2r, relu=False)
    return out[:N]
